# baseline (speedup 1.0000x reference)
import functools

import jax
import jax.numpy as jnp
from jax import lax
from jax.experimental import pallas as pl
from jax.experimental.pallas import tpu as pltpu
from jax.experimental.pallas import tpu_sc as plsc

N = 10000
D = 128
E = 320000
NPAD = 10240
NC = 2
NS = 16
TILES = NC * NS
EB = 112
NBLK = 96
NPART = 4
PBLK = NBLK // NPART
EPAD = TILES * NBLK * EB
RPT = NPAD // NS
DEGW = 16


def _deg_body(dst_hbm, out_hbm, idx_v, ones_v, acc, sem):
    c = lax.axis_index("c")
    s = lax.axis_index("s")
    wid = s * NC + c

    @pl.loop(0, EB)
    def _(r):
        ones_v[r, :] = jnp.full((DEGW,), 1.0, jnp.float32)

    @pl.loop(0, RPT // (2 * EB))
    def _(k):
        pltpu.sync_copy(ones_v, acc.at[pl.ds(s * RPT + (2 * k) * EB, EB)])
        pltpu.sync_copy(ones_v, acc.at[pl.ds(s * RPT + (2 * k + 1) * EB, EB)])

    pltpu.async_copy(dst_hbm.at[wid], idx_v, sem).wait()
    plsc.subcore_barrier()

    @pl.loop(0, NBLK)
    def _(j):
        pltpu.sync_copy(ones_v, acc.at[idx_v.at[j]], add=True)

    plsc.subcore_barrier()
    pltpu.sync_copy(acc.at[pl.ds(s * RPT, RPT)], out_hbm.at[c, pl.ds(s * RPT, RPT)])


def _agg_body(hp_hbm, src_hbm, dst_hbm, out_hbm,
              si_v, di_v, buf0, buf1, buf2,
              acc, gsem0, gsem1, gsem2, ssem0, ssem1, ssem2, csem):
    c = lax.axis_index("c")
    s = lax.axis_index("s")
    wid = s * NC + c
    bufs = (buf0, buf1, buf2)
    gsems = (gsem0, gsem1, gsem2)
    ssems = (ssem0, ssem1, ssem2)

    pltpu.sync_copy(hp_hbm.at[pl.ds(s * RPT, RPT)], acc.at[pl.ds(s * RPT, RPT)])

    plsc.subcore_barrier()

    for part in range(NPART):
        pltpu.async_copy(src_hbm.at[wid, pl.ds(part * PBLK, PBLK)], si_v, csem).wait()
        pltpu.async_copy(dst_hbm.at[wid, pl.ds(part * PBLK, PBLK)], di_v, csem).wait()
        for b in range(3):
            pltpu.async_copy(hp_hbm.at[si_v.at[b]], bufs[b], gsems[b])

        @pl.loop(0, PBLK, step=3)
        def _(j):
            for b in range(3):
                pltpu.make_async_copy(hp_hbm.at[si_v.at[j + b]], bufs[b], gsems[b]).wait()
                pltpu.async_copy(bufs[b], acc.at[di_v.at[j + b]], ssems[b], add=True)
            for b in range(3):
                @pl.when(j + 3 + b < PBLK)
                def _(b=b):
                    pltpu.make_async_copy(bufs[b], acc.at[di_v.at[j + b]], ssems[b]).wait()
                    pltpu.async_copy(hp_hbm.at[si_v.at[j + 3 + b]], bufs[b], gsems[b])

        for b in range(3):
            pltpu.make_async_copy(bufs[b], acc.at[di_v.at[PBLK - 3 + b]], ssems[b]).wait()

    plsc.subcore_barrier()

    pltpu.sync_copy(acc.at[pl.ds(s * RPT, RPT)], out_hbm.at[c, pl.ds(s * RPT, RPT)])


@functools.cache
def _sc_kernels():
    mesh = plsc.VectorSubcoreMesh(
        core_axis_name="c", subcore_axis_name="s", num_cores=NC, num_subcores=NS
    )
    deg_kernel = pl.kernel(
        _deg_body,
        out_type=jax.ShapeDtypeStruct((NC, NPAD, DEGW), jnp.float32),
        mesh=mesh,
        scratch_types=[
            pltpu.VMEM((NBLK, EB), jnp.int32),
            pltpu.VMEM((EB, DEGW), jnp.float32),
            pltpu.VMEM_SHARED((NPAD, DEGW), jnp.float32),
            pltpu.SemaphoreType.DMA,
        ],
    )
    agg_kernel = pl.kernel(
        _agg_body,
        out_type=jax.ShapeDtypeStruct((NC, NPAD, D), jnp.float32),
        mesh=mesh,
        scratch_types=[
            pltpu.VMEM((PBLK, EB), jnp.int32),
            pltpu.VMEM((PBLK, EB), jnp.int32),
            pltpu.VMEM((EB, D), jnp.float32),
            pltpu.VMEM((EB, D), jnp.float32),
            pltpu.VMEM((EB, D), jnp.float32),
            pltpu.VMEM_SHARED((NPAD, D), jnp.float32),
            pltpu.SemaphoreType.DMA,
            pltpu.SemaphoreType.DMA,
            pltpu.SemaphoreType.DMA,
            pltpu.SemaphoreType.DMA,
            pltpu.SemaphoreType.DMA,
            pltpu.SemaphoreType.DMA,
            pltpu.SemaphoreType.DMA,
        ],
    )
    return deg_kernel, agg_kernel


ER = E // 128
EPR = EPAD // 128
EBLK = 128


def _pad_body(e_ref, o_ref):
    g = pl.program_id(0)
    i = pl.program_id(1)
    flat = (i * EBLK * 128
            + lax.broadcasted_iota(jnp.int32, (EBLK, 128), 0) * 128
            + lax.broadcasted_iota(jnp.int32, (EBLK, 128), 1))
    padv = N + lax.rem(flat - E, NPAD - N)
    o_ref[0] = jnp.where(flat < E, e_ref[0], padv)


_pad_tc = pl.pallas_call(
    _pad_body,
    grid=(2, EPR // EBLK),
    in_specs=[
        pl.BlockSpec((1, EBLK, 128), lambda g, i: (g, jnp.minimum(i, ER // EBLK), 0)),
    ],
    out_specs=pl.BlockSpec((1, EBLK, 128), lambda g, i: (g, i, 0)),
    out_shape=jax.ShapeDtypeStruct((2, EPR, 128), jnp.int32),
)


RB = 2048
RBF = 2000


def _dis_block(da_ref, db_ref, i, rb, mask):
    deg = da_ref[0, :, :1] + db_ref[0, :, :1] - 1.0
    dis = lax.rsqrt(deg)
    if not mask:
        return dis
    row = i * rb + lax.broadcasted_iota(jnp.int32, (rb, 1), 0)
    return jnp.where(row < N, dis, 0.0)


def _first_body(x_ref, w_ref, da_ref, db_ref, o_ref):
    i = pl.program_id(0)
    dis = _dis_block(da_ref, db_ref, i, RB, True)
    row = i * RB + lax.broadcasted_iota(jnp.int32, (RB, 1), 0)
    xq = jnp.where(row < N, x_ref[...], 0.0)
    o_ref[...] = (
        jnp.dot(xq, w_ref[...], preferred_element_type=jnp.float32) * dis
    )


_first_tc = pl.pallas_call(
    _first_body,
    grid=(NPAD // RB,),
    in_specs=[
        pl.BlockSpec((RB, D), lambda i: (i, 0)),
        pl.BlockSpec((D, D), lambda i: (0, 0)),
        pl.BlockSpec((1, RB, DEGW), lambda i: (0, i, 0)),
        pl.BlockSpec((1, RB, DEGW), lambda i: (1, i, 0)),
    ],
    out_specs=pl.BlockSpec((RB, D), lambda i: (i, 0)),
    out_shape=jax.ShapeDtypeStruct((NPAD, D), jnp.float32),
)


def _mid_body(a0_ref, a1_ref, hp_ref, da_ref, db_ref, b_ref, w_ref, o_ref):
    dis = _dis_block(da_ref, db_ref, pl.program_id(0), RB, True)
    z = (a0_ref[0] + a1_ref[0] - hp_ref[...]) * dis + b_ref[...]
    xr = jnp.maximum(z, 0.0)
    o_ref[...] = (
        jnp.dot(xr, w_ref[...], preferred_element_type=jnp.float32) * dis
    )


_mid_tc = pl.pallas_call(
    _mid_body,
    grid=(NPAD // RB,),
    in_specs=[
        pl.BlockSpec((1, RB, D), lambda i: (0, i, 0)),
        pl.BlockSpec((1, RB, D), lambda i: (1, i, 0)),
        pl.BlockSpec((RB, D), lambda i: (i, 0)),
        pl.BlockSpec((1, RB, DEGW), lambda i: (0, i, 0)),
        pl.BlockSpec((1, RB, DEGW), lambda i: (1, i, 0)),
        pl.BlockSpec((1, D), lambda i: (0, 0)),
        pl.BlockSpec((D, D), lambda i: (0, 0)),
    ],
    out_specs=pl.BlockSpec((RB, D), lambda i: (i, 0)),
    out_shape=jax.ShapeDtypeStruct((NPAD, D), jnp.float32),
)


def _final_body(a0_ref, a1_ref, hp_ref, da_ref, db_ref, b_ref, o_ref):
    dis = _dis_block(da_ref, db_ref, pl.program_id(0), RBF, False)
    z = (a0_ref[0] + a1_ref[0] - hp_ref[...]) * dis + b_ref[...]
    xr = jnp.maximum(z, 0.0)
    m = jnp.max(xr, axis=1, keepdims=True)
    y = xr - m
    lse = jnp.log(jnp.sum(jnp.exp(y), axis=1, keepdims=True))
    o_ref[...] = y - lse


_final_tc = pl.pallas_call(
    _final_body,
    grid=(N // RBF,),
    in_specs=[
        pl.BlockSpec((1, RBF, D), lambda i: (0, i, 0)),
        pl.BlockSpec((1, RBF, D), lambda i: (1, i, 0)),
        pl.BlockSpec((RBF, D), lambda i: (i, 0)),
        pl.BlockSpec((1, RBF, DEGW), lambda i: (0, i, 0)),
        pl.BlockSpec((1, RBF, DEGW), lambda i: (1, i, 0)),
        pl.BlockSpec((1, D), lambda i: (0, 0)),
    ],
    out_specs=pl.BlockSpec((RBF, D), lambda i: (i, 0)),
    out_shape=jax.ShapeDtypeStruct((N, D), jnp.float32),
)


def kernel(x, edge_index, W1, b1, W2, b2, W3, b3):
    ei = edge_index.astype(jnp.int32).reshape(2, ER, 128)
    ei_p = _pad_tc(ei)
    src_p = ei_p[0].reshape(TILES, NBLK, EB)
    dst_p = ei_p[1].reshape(TILES, NBLK, EB)

    _deg_kernel, _agg_kernel = _sc_kernels()
    degp = _deg_kernel(dst_p)
    b1r = b1.reshape(1, D)
    b2r = b2.reshape(1, D)
    b3r = b3.reshape(1, D)

    hp1 = _first_tc(x, W1, degp, degp)
    agg1 = _agg_kernel(hp1, src_p, dst_p)
    hp2 = _mid_tc(agg1, agg1, hp1, degp, degp, b1r, W2)
    agg2 = _agg_kernel(hp2, src_p, dst_p)
    hp3 = _mid_tc(agg2, agg2, hp2, degp, degp, b2r, W3)
    agg3 = _agg_kernel(hp3, src_p, dst_p)
    return _final_tc(agg3, agg3, hp3, degp, degp, b3r)


# speedup vs baseline: 1.0283x; 1.0283x over previous
import functools

import jax
import jax.numpy as jnp
from jax import lax
from jax.experimental import pallas as pl
from jax.experimental.pallas import tpu as pltpu
from jax.experimental.pallas import tpu_sc as plsc

N = 10000
D = 128
E = 320000
NPAD = 10240
NC = 2
NS = 16
TILES = NC * NS
EB = 112
NBLK = 96
NPART = 4
PBLK = NBLK // NPART
EPAD = TILES * NBLK * EB
RPT = NPAD // NS
DEGW = 16


def _deg_body(dst_hbm, out_hbm, idx_v, ones_v, acc, sem):
    c = lax.axis_index("c")
    s = lax.axis_index("s")
    wid = s * NC + c

    @pl.loop(0, EB)
    def _(r):
        ones_v[r, :] = jnp.full((DEGW,), 1.0, jnp.float32)

    @pl.loop(0, RPT // (2 * EB))
    def _(k):
        pltpu.sync_copy(ones_v, acc.at[pl.ds(s * RPT + (2 * k) * EB, EB)])
        pltpu.sync_copy(ones_v, acc.at[pl.ds(s * RPT + (2 * k + 1) * EB, EB)])

    pltpu.async_copy(dst_hbm.at[wid], idx_v, sem).wait()
    plsc.subcore_barrier()

    @pl.loop(0, NBLK)
    def _(j):
        pltpu.sync_copy(ones_v, acc.at[idx_v.at[j]], add=True)

    plsc.subcore_barrier()
    pltpu.sync_copy(acc.at[pl.ds(s * RPT, RPT)], out_hbm.at[c, pl.ds(s * RPT, RPT)])


def _agg_body(hp_hbm, src_hbm, dst_hbm, out_hbm,
              si_v, di_v, buf0, buf1, buf2,
              acc, gsem0, gsem1, gsem2, ssem0, ssem1, ssem2, csem):
    c = lax.axis_index("c")
    s = lax.axis_index("s")
    wid = s * NC + c
    bufs = (buf0, buf1, buf2)
    gsems = (gsem0, gsem1, gsem2)
    ssems = (ssem0, ssem1, ssem2)

    pltpu.sync_copy(hp_hbm.at[pl.ds(s * RPT, RPT)], acc.at[pl.ds(s * RPT, RPT)])

    plsc.subcore_barrier()

    for part in range(NPART):
        pltpu.async_copy(src_hbm.at[wid, pl.ds(part * PBLK, PBLK)], si_v, csem).wait()
        pltpu.async_copy(dst_hbm.at[wid, pl.ds(part * PBLK, PBLK)], di_v, csem).wait()
        for b in range(3):
            pltpu.async_copy(hp_hbm.at[si_v.at[b]], bufs[b], gsems[b])

        @pl.loop(0, PBLK, step=3)
        def _(j):
            for b in range(3):
                pltpu.make_async_copy(hp_hbm.at[si_v.at[j + b]], bufs[b], gsems[b]).wait()
                pltpu.async_copy(bufs[b], acc.at[di_v.at[j + b]], ssems[b], add=True)
            for b in range(3):
                @pl.when(j + 3 + b < PBLK)
                def _(b=b):
                    pltpu.make_async_copy(bufs[b], acc.at[di_v.at[j + b]], ssems[b]).wait()
                    pltpu.async_copy(hp_hbm.at[si_v.at[j + 3 + b]], bufs[b], gsems[b])

        for b in range(3):
            pltpu.make_async_copy(bufs[b], acc.at[di_v.at[PBLK - 3 + b]], ssems[b]).wait()

    plsc.subcore_barrier()

    pltpu.sync_copy(acc.at[pl.ds(s * RPT, RPT)], out_hbm.at[c, pl.ds(s * RPT, RPT)])


@functools.cache
def _sc_kernels():
    mesh = plsc.VectorSubcoreMesh(
        core_axis_name="c", subcore_axis_name="s", num_cores=NC, num_subcores=NS
    )
    deg_kernel = pl.kernel(
        _deg_body,
        out_type=jax.ShapeDtypeStruct((NC, NPAD, DEGW), jnp.float32),
        mesh=mesh,
        scratch_types=[
            pltpu.VMEM((NBLK, EB), jnp.int32),
            pltpu.VMEM((EB, DEGW), jnp.float32),
            pltpu.VMEM_SHARED((NPAD, DEGW), jnp.float32),
            pltpu.SemaphoreType.DMA,
        ],
    )
    agg_kernel = pl.kernel(
        _agg_body,
        out_type=jax.ShapeDtypeStruct((NC, NPAD, D), jnp.float32),
        mesh=mesh,
        scratch_types=[
            pltpu.VMEM((PBLK, EB), jnp.int32),
            pltpu.VMEM((PBLK, EB), jnp.int32),
            pltpu.VMEM((EB, D), jnp.float32),
            pltpu.VMEM((EB, D), jnp.float32),
            pltpu.VMEM((EB, D), jnp.float32),
            pltpu.VMEM_SHARED((NPAD, D), jnp.float32),
            pltpu.SemaphoreType.DMA,
            pltpu.SemaphoreType.DMA,
            pltpu.SemaphoreType.DMA,
            pltpu.SemaphoreType.DMA,
            pltpu.SemaphoreType.DMA,
            pltpu.SemaphoreType.DMA,
            pltpu.SemaphoreType.DMA,
        ],
    )
    return deg_kernel, agg_kernel


RB = 2048
RBF = 2000


def _dis_block(da_ref, db_ref, i, rb, mask):
    deg = da_ref[0, :, :1] + db_ref[0, :, :1] - 1.0
    dis = lax.rsqrt(deg)
    if not mask:
        return dis
    row = i * rb + lax.broadcasted_iota(jnp.int32, (rb, 1), 0)
    return jnp.where(row < N, dis, 0.0)


def _first_body(x_ref, w_ref, da_ref, db_ref, o_ref):
    i = pl.program_id(0)
    dis = _dis_block(da_ref, db_ref, i, RB, True)
    row = i * RB + lax.broadcasted_iota(jnp.int32, (RB, 1), 0)
    xq = jnp.where(row < N, x_ref[...], 0.0)
    o_ref[...] = (
        jnp.dot(xq, w_ref[...], preferred_element_type=jnp.float32) * dis
    )


_first_tc = pl.pallas_call(
    _first_body,
    grid=(NPAD // RB,),
    in_specs=[
        pl.BlockSpec((RB, D), lambda i: (i, 0)),
        pl.BlockSpec((D, D), lambda i: (0, 0)),
        pl.BlockSpec((1, RB, DEGW), lambda i: (0, i, 0)),
        pl.BlockSpec((1, RB, DEGW), lambda i: (1, i, 0)),
    ],
    out_specs=pl.BlockSpec((RB, D), lambda i: (i, 0)),
    out_shape=jax.ShapeDtypeStruct((NPAD, D), jnp.float32),
)


def _mid_body(a0_ref, a1_ref, hp_ref, da_ref, db_ref, b_ref, w_ref, o_ref):
    dis = _dis_block(da_ref, db_ref, pl.program_id(0), RB, True)
    z = (a0_ref[0] + a1_ref[0] - hp_ref[...]) * dis + b_ref[...]
    xr = jnp.maximum(z, 0.0)
    o_ref[...] = (
        jnp.dot(xr, w_ref[...], preferred_element_type=jnp.float32) * dis
    )


_mid_tc = pl.pallas_call(
    _mid_body,
    grid=(NPAD // RB,),
    in_specs=[
        pl.BlockSpec((1, RB, D), lambda i: (0, i, 0)),
        pl.BlockSpec((1, RB, D), lambda i: (1, i, 0)),
        pl.BlockSpec((RB, D), lambda i: (i, 0)),
        pl.BlockSpec((1, RB, DEGW), lambda i: (0, i, 0)),
        pl.BlockSpec((1, RB, DEGW), lambda i: (1, i, 0)),
        pl.BlockSpec((1, D), lambda i: (0, 0)),
        pl.BlockSpec((D, D), lambda i: (0, 0)),
    ],
    out_specs=pl.BlockSpec((RB, D), lambda i: (i, 0)),
    out_shape=jax.ShapeDtypeStruct((NPAD, D), jnp.float32),
)


def _final_body(a0_ref, a1_ref, hp_ref, da_ref, db_ref, b_ref, o_ref):
    dis = _dis_block(da_ref, db_ref, pl.program_id(0), RBF, False)
    z = (a0_ref[0] + a1_ref[0] - hp_ref[...]) * dis + b_ref[...]
    xr = jnp.maximum(z, 0.0)
    m = jnp.max(xr, axis=1, keepdims=True)
    y = xr - m
    lse = jnp.log(jnp.sum(jnp.exp(y), axis=1, keepdims=True))
    o_ref[...] = y - lse


_final_tc = pl.pallas_call(
    _final_body,
    grid=(N // RBF,),
    in_specs=[
        pl.BlockSpec((1, RBF, D), lambda i: (0, i, 0)),
        pl.BlockSpec((1, RBF, D), lambda i: (1, i, 0)),
        pl.BlockSpec((RBF, D), lambda i: (i, 0)),
        pl.BlockSpec((1, RBF, DEGW), lambda i: (0, i, 0)),
        pl.BlockSpec((1, RBF, DEGW), lambda i: (1, i, 0)),
        pl.BlockSpec((1, D), lambda i: (0, 0)),
    ],
    out_specs=pl.BlockSpec((RBF, D), lambda i: (i, 0)),
    out_shape=jax.ShapeDtypeStruct((N, D), jnp.float32),
)


def kernel(x, edge_index, W1, b1, W2, b2, W3, b3):
    src = edge_index[0].astype(jnp.int32)
    dst = edge_index[1].astype(jnp.int32)
    pad = N + (jnp.arange(EPAD - E, dtype=jnp.int32) % (NPAD - N))
    src_p = jnp.concatenate([src, pad]).reshape(TILES, NBLK, EB)
    dst_p = jnp.concatenate([dst, pad]).reshape(TILES, NBLK, EB)

    _deg_kernel, _agg_kernel = _sc_kernels()
    degp = _deg_kernel(dst_p)
    b1r = b1.reshape(1, D)
    b2r = b2.reshape(1, D)
    b3r = b3.reshape(1, D)

    hp1 = _first_tc(x, W1, degp, degp)
    agg1 = _agg_kernel(hp1, src_p, dst_p)
    hp2 = _mid_tc(agg1, agg1, hp1, degp, degp, b1r, W2)
    agg2 = _agg_kernel(hp2, src_p, dst_p)
    hp3 = _mid_tc(agg2, agg2, hp2, degp, degp, b2r, W3)
    agg3 = _agg_kernel(hp3, src_p, dst_p)
    return _final_tc(agg3, agg3, hp3, degp, degp, b3r)


# speedup vs baseline: 1.1244x; 1.0935x over previous
import functools

import jax
import jax.numpy as jnp
from jax import lax
from jax.experimental import pallas as pl
from jax.experimental.pallas import tpu as pltpu
from jax.experimental.pallas import tpu_sc as plsc

N = 10000
D = 128
E = 320000
NPAD = 10240
NC = 2
NS = 16
TILES = NC * NS
EB = 80
NBLK = 128
NPART = 4
PBLK = NBLK // NPART
EPAD = TILES * NBLK * EB
RPT = NPAD // NS
DEGW = 16


def _deg_body(dst_hbm, out_hbm, idx_v, ones_v, acc, sem):
    c = lax.axis_index("c")
    s = lax.axis_index("s")
    wid = s * NC + c

    @pl.loop(0, EB)
    def _(r):
        ones_v[r, :] = jnp.full((DEGW,), 1.0, jnp.float32)

    @pl.loop(0, RPT // (2 * EB))
    def _(k):
        pltpu.sync_copy(ones_v, acc.at[pl.ds(s * RPT + (2 * k) * EB, EB)])
        pltpu.sync_copy(ones_v, acc.at[pl.ds(s * RPT + (2 * k + 1) * EB, EB)])

    pltpu.async_copy(dst_hbm.at[wid], idx_v, sem).wait()
    plsc.subcore_barrier()

    @pl.loop(0, NBLK)
    def _(j):
        pltpu.sync_copy(ones_v, acc.at[idx_v.at[j]], add=True)

    plsc.subcore_barrier()
    pltpu.sync_copy(acc.at[pl.ds(s * RPT, RPT)], out_hbm.at[c, pl.ds(s * RPT, RPT)])


def _agg_body(hp_hbm, src_hbm, dst_hbm, out_hbm,
              si_v, di_v, buf0, buf1, buf2, buf3,
              acc, gsem0, gsem1, gsem2, gsem3, ssem0, ssem1, ssem2, ssem3, csem):
    c = lax.axis_index("c")
    s = lax.axis_index("s")
    wid = s * NC + c
    bufs = (buf0, buf1, buf2, buf3)
    gsems = (gsem0, gsem1, gsem2, gsem3)
    ssems = (ssem0, ssem1, ssem2, ssem3)

    pltpu.sync_copy(hp_hbm.at[pl.ds(s * RPT, RPT)], acc.at[pl.ds(s * RPT, RPT)])

    plsc.subcore_barrier()

    for part in range(NPART):
        pltpu.async_copy(src_hbm.at[wid, pl.ds(part * PBLK, PBLK)], si_v, csem).wait()
        pltpu.async_copy(dst_hbm.at[wid, pl.ds(part * PBLK, PBLK)], di_v, csem).wait()
        for b in range(4):
            pltpu.async_copy(hp_hbm.at[si_v.at[b]], bufs[b], gsems[b])

        @pl.loop(0, PBLK, step=4)
        def _(j):
            for b in range(4):
                pltpu.make_async_copy(hp_hbm.at[si_v.at[j + b]], bufs[b], gsems[b]).wait()
                pltpu.async_copy(bufs[b], acc.at[di_v.at[j + b]], ssems[b], add=True)
            for b in range(4):
                @pl.when(j + 4 + b < PBLK)
                def _(b=b):
                    pltpu.make_async_copy(bufs[b], acc.at[di_v.at[j + b]], ssems[b]).wait()
                    pltpu.async_copy(hp_hbm.at[si_v.at[j + 4 + b]], bufs[b], gsems[b])

        for b in range(4):
            pltpu.make_async_copy(bufs[b], acc.at[di_v.at[PBLK - 4 + b]], ssems[b]).wait()

    plsc.subcore_barrier()

    pltpu.sync_copy(acc.at[pl.ds(s * RPT, RPT)], out_hbm.at[c, pl.ds(s * RPT, RPT)])


@functools.cache
def _sc_kernels():
    mesh = plsc.VectorSubcoreMesh(
        core_axis_name="c", subcore_axis_name="s", num_cores=NC, num_subcores=NS
    )
    deg_kernel = pl.kernel(
        _deg_body,
        out_type=jax.ShapeDtypeStruct((NC, NPAD, DEGW), jnp.float32),
        mesh=mesh,
        scratch_types=[
            pltpu.VMEM((NBLK, EB), jnp.int32),
            pltpu.VMEM((EB, DEGW), jnp.float32),
            pltpu.VMEM_SHARED((NPAD, DEGW), jnp.float32),
            pltpu.SemaphoreType.DMA,
        ],
    )
    agg_kernel = pl.kernel(
        _agg_body,
        out_type=jax.ShapeDtypeStruct((NC, NPAD, D), jnp.float32),
        mesh=mesh,
        scratch_types=[
            pltpu.VMEM((PBLK, EB), jnp.int32),
            pltpu.VMEM((PBLK, EB), jnp.int32),
            pltpu.VMEM((EB, D), jnp.float32),
            pltpu.VMEM((EB, D), jnp.float32),
            pltpu.VMEM((EB, D), jnp.float32),
            pltpu.VMEM((EB, D), jnp.float32),
            pltpu.VMEM_SHARED((NPAD, D), jnp.float32),
            pltpu.SemaphoreType.DMA,
            pltpu.SemaphoreType.DMA,
            pltpu.SemaphoreType.DMA,
            pltpu.SemaphoreType.DMA,
            pltpu.SemaphoreType.DMA,
            pltpu.SemaphoreType.DMA,
            pltpu.SemaphoreType.DMA,
            pltpu.SemaphoreType.DMA,
            pltpu.SemaphoreType.DMA,
        ],
    )
    return deg_kernel, agg_kernel


RB = 2048
RBF = 2000


def _dis_block(da_ref, db_ref, i, rb, mask):
    deg = da_ref[0, :, :1] + db_ref[0, :, :1] - 1.0
    dis = lax.rsqrt(deg)
    if not mask:
        return dis
    row = i * rb + lax.broadcasted_iota(jnp.int32, (rb, 1), 0)
    return jnp.where(row < N, dis, 0.0)


def _first_body(x_ref, w_ref, da_ref, db_ref, o_ref):
    i = pl.program_id(0)
    dis = _dis_block(da_ref, db_ref, i, RB, True)
    row = i * RB + lax.broadcasted_iota(jnp.int32, (RB, 1), 0)
    xq = jnp.where(row < N, x_ref[...], 0.0)
    o_ref[...] = (
        jnp.dot(xq, w_ref[...], preferred_element_type=jnp.float32) * dis
    )


_first_tc = pl.pallas_call(
    _first_body,
    grid=(NPAD // RB,),
    in_specs=[
        pl.BlockSpec((RB, D), lambda i: (i, 0)),
        pl.BlockSpec((D, D), lambda i: (0, 0)),
        pl.BlockSpec((1, RB, DEGW), lambda i: (0, i, 0)),
        pl.BlockSpec((1, RB, DEGW), lambda i: (1, i, 0)),
    ],
    out_specs=pl.BlockSpec((RB, D), lambda i: (i, 0)),
    out_shape=jax.ShapeDtypeStruct((NPAD, D), jnp.float32),
)


def _mid_body(a0_ref, a1_ref, hp_ref, da_ref, db_ref, b_ref, w_ref, o_ref):
    dis = _dis_block(da_ref, db_ref, pl.program_id(0), RB, True)
    z = (a0_ref[0] + a1_ref[0] - hp_ref[...]) * dis + b_ref[...]
    xr = jnp.maximum(z, 0.0)
    o_ref[...] = (
        jnp.dot(xr, w_ref[...], preferred_element_type=jnp.float32) * dis
    )


_mid_tc = pl.pallas_call(
    _mid_body,
    grid=(NPAD // RB,),
    in_specs=[
        pl.BlockSpec((1, RB, D), lambda i: (0, i, 0)),
        pl.BlockSpec((1, RB, D), lambda i: (1, i, 0)),
        pl.BlockSpec((RB, D), lambda i: (i, 0)),
        pl.BlockSpec((1, RB, DEGW), lambda i: (0, i, 0)),
        pl.BlockSpec((1, RB, DEGW), lambda i: (1, i, 0)),
        pl.BlockSpec((1, D), lambda i: (0, 0)),
        pl.BlockSpec((D, D), lambda i: (0, 0)),
    ],
    out_specs=pl.BlockSpec((RB, D), lambda i: (i, 0)),
    out_shape=jax.ShapeDtypeStruct((NPAD, D), jnp.float32),
)


def _final_body(a0_ref, a1_ref, hp_ref, da_ref, db_ref, b_ref, o_ref):
    dis = _dis_block(da_ref, db_ref, pl.program_id(0), RBF, False)
    z = (a0_ref[0] + a1_ref[0] - hp_ref[...]) * dis + b_ref[...]
    xr = jnp.maximum(z, 0.0)
    m = jnp.max(xr, axis=1, keepdims=True)
    y = xr - m
    lse = jnp.log(jnp.sum(jnp.exp(y), axis=1, keepdims=True))
    o_ref[...] = y - lse


_final_tc = pl.pallas_call(
    _final_body,
    grid=(N // RBF,),
    in_specs=[
        pl.BlockSpec((1, RBF, D), lambda i: (0, i, 0)),
        pl.BlockSpec((1, RBF, D), lambda i: (1, i, 0)),
        pl.BlockSpec((RBF, D), lambda i: (i, 0)),
        pl.BlockSpec((1, RBF, DEGW), lambda i: (0, i, 0)),
        pl.BlockSpec((1, RBF, DEGW), lambda i: (1, i, 0)),
        pl.BlockSpec((1, D), lambda i: (0, 0)),
    ],
    out_specs=pl.BlockSpec((RBF, D), lambda i: (i, 0)),
    out_shape=jax.ShapeDtypeStruct((N, D), jnp.float32),
)


def kernel(x, edge_index, W1, b1, W2, b2, W3, b3):
    src = edge_index[0].astype(jnp.int32)
    dst = edge_index[1].astype(jnp.int32)
    pad = N + (jnp.arange(EPAD - E, dtype=jnp.int32) % (NPAD - N))
    src_p = jnp.concatenate([src, pad]).reshape(TILES, NBLK, EB)
    dst_p = jnp.concatenate([dst, pad]).reshape(TILES, NBLK, EB)

    _deg_kernel, _agg_kernel = _sc_kernels()
    degp = _deg_kernel(dst_p)
    b1r = b1.reshape(1, D)
    b2r = b2.reshape(1, D)
    b3r = b3.reshape(1, D)

    hp1 = _first_tc(x, W1, degp, degp)
    agg1 = _agg_kernel(hp1, src_p, dst_p)
    hp2 = _mid_tc(agg1, agg1, hp1, degp, degp, b1r, W2)
    agg2 = _agg_kernel(hp2, src_p, dst_p)
    hp3 = _mid_tc(agg2, agg2, hp2, degp, degp, b2r, W3)
    agg3 = _agg_kernel(hp3, src_p, dst_p)
    return _final_tc(agg3, agg3, hp3, degp, degp, b3r)
